# Initial kernel scaffold; baseline (speedup 1.0000x reference)
import jax
import jax.numpy as jnp
from jax.experimental import pallas as pl


def kernel(features, W):
    raise NotImplementedError("write your pallas kernel here")


# speedup vs baseline: 174.5800x; 174.5800x over previous
import functools

import jax
import jax.numpy as jnp
from jax import lax
from jax.experimental import pallas as pl
from jax.experimental.pallas import tpu as pltpu
from jax.experimental.pallas import tpu_sc as plsc

_N_CHANNELS = 100
_RESOLUTIONS = (16, 64, 256, 1024)
_OFFSETS = (0, 1700, 8200, 33900)
_DIM = 64
_BATCH = 4096
_NC, _NS, _L = 2, 16, 16
_NW = _NC * _NS
_BPW = _BATCH // _NW
_NIDX = _N_CHANNELS * len(_RESOLUTIONS)
_MB = _BPW // _L


def _bucket_ids(v, r):
    _ONE = jnp.full((_L,), 1, jnp.int32)
    _ZERO = jnp.full((_L,), 0, jnp.int32)
    scale = jnp.float32(r - 1)
    delta = jnp.float32(1.0) / jnp.float32(r - 1)
    t = v * scale
    f = t.astype(jnp.int32)
    k = f + jnp.where(t > f.astype(jnp.float32), _ONE, _ZERO)
    k = jnp.minimum(k, r - 1)
    km1 = k - 1
    b_lo = jnp.where(km1 < 0, jnp.float32(-1.0),
                     km1.astype(jnp.float32) * delta)
    b_hi = jnp.where(k == r - 1, jnp.float32(1.0),
                     k.astype(jnp.float32) * delta)
    return (km1 + jnp.where(b_lo < v, _ONE, _ZERO)
            + jnp.where(b_hi < v, _ONE, _ZERO))


_MESH = plsc.VectorSubcoreMesh(core_axis_name="c", subcore_axis_name="s",
                               num_cores=_NC, num_subcores=_NS)


@functools.partial(
    pl.kernel,
    out_type=jax.ShapeDtypeStruct((_BATCH, _DIM), jnp.float32),
    mesh=_MESH,
    compiler_params=pltpu.CompilerParams(use_tc_tiling_on_sc=False),
    scratch_types=[
        pltpu.VMEM((_N_CHANNELS, _BPW), jnp.float32),
        pltpu.VMEM((_NIDX, _BPW), jnp.int32),
        pltpu.VMEM((_BPW, _DIM), jnp.float32),
        pltpu.VMEM((_BPW, _DIM), jnp.float32),
        pltpu.VMEM((_BPW, _DIM), jnp.float32),
        pltpu.SemaphoreType.DMA,
        pltpu.SemaphoreType.DMA,
        pltpu.SemaphoreType.DMA,
    ],
)
def _emb(ft_hbm, w_hbm, out_hbm, feat_v, idx_v, acc_v, buf0, buf1,
         sem_f, sem0, sem1):
    wid = lax.axis_index("s") * _NC + lax.axis_index("c")
    base = wid * _BPW

    pltpu.async_copy(ft_hbm.at[:, pl.ds(base, _BPW)], feat_v, sem_f).wait()

    zeros = jnp.zeros((_L,), jnp.float32)

    @pl.loop(0, _N_CHANNELS)
    def _phase_idx(c):
        for i, r in enumerate(_RESOLUTIONS):
            off = c * (r + 1) + _OFFSETS[i]
            row = i * _N_CHANNELS + c
            for m in range(_MB):
                v = feat_v[c, pl.ds(m * _L, _L)]
                idx_v[row, pl.ds(m * _L, _L)] = _bucket_ids(v, r) + off

    @pl.loop(0, _BPW, unroll=8)
    def _phase_zero(b):
        for q in range(_DIM // _L):
            acc_v[b, pl.ds(q * _L, _L)] = zeros

    def _fire(j, buf, sem):
        pltpu.async_copy(w_hbm.at[idx_v.at[j]], buf, sem)

    def _wait(j, buf, sem):
        pltpu.make_async_copy(w_hbm.at[idx_v.at[j]], buf, sem).wait()

    def _accum(buf):
        @pl.loop(0, _BPW, unroll=8)
        def _(b):
            for q in range(_DIM // _L):
                plsc.addupdate(acc_v.at[b, pl.ds(q * _L, _L)],
                               buf[b, pl.ds(q * _L, _L)])

    _fire(0, buf0, sem0)
    _fire(1, buf1, sem1)

    @pl.loop(0, _NIDX, step=2)
    def _phase_gather(j):
        _wait(j, buf0, sem0)
        _accum(buf0)

        @pl.when(j + 2 < _NIDX)
        def _():
            _fire(j + 2, buf0, sem0)

        _wait(j + 1, buf1, sem1)
        _accum(buf1)

        @pl.when(j + 3 < _NIDX)
        def _():
            _fire(j + 3, buf1, sem1)

    pltpu.sync_copy(acc_v, out_hbm.at[pl.ds(base, _BPW), :])


def kernel(features, W):
    return _emb(features.T, W)


# speedup vs baseline: 240.0137x; 1.3748x over previous
import functools

import jax
import jax.numpy as jnp
from jax import lax
from jax.experimental import pallas as pl
from jax.experimental.pallas import tpu as pltpu
from jax.experimental.pallas import tpu_sc as plsc

_N_CHANNELS = 100
_RESOLUTIONS = (16, 64, 256, 1024)
_OFFSETS = (0, 1700, 8200, 33900)
_DIM = 64
_BATCH = 4096
_NC, _NS, _L = 2, 16, 16
_NW = _NC * _NS
_BPW = _BATCH // _NW
_NIDX = _N_CHANNELS * len(_RESOLUTIONS)
_MB = _BPW // _L
_KWIN = 8


def _bucket_ids(v, r):
    _ONE = jnp.full((_L,), 1, jnp.int32)
    _ZERO = jnp.full((_L,), 0, jnp.int32)
    scale = jnp.float32(r - 1)
    delta = jnp.float32(1.0) / jnp.float32(r - 1)
    t = v * scale
    f = t.astype(jnp.int32)
    k = f + jnp.where(t > f.astype(jnp.float32), _ONE, _ZERO)
    k = jnp.minimum(k, r - 1)
    km1 = k - 1
    b_lo = jnp.where(km1 < 0, jnp.float32(-1.0),
                     km1.astype(jnp.float32) * delta)
    b_hi = jnp.where(k == r - 1, jnp.float32(1.0),
                     k.astype(jnp.float32) * delta)
    return (km1 + jnp.where(b_lo < v, _ONE, _ZERO)
            + jnp.where(b_hi < v, _ONE, _ZERO))


_MESH = plsc.VectorSubcoreMesh(core_axis_name="c", subcore_axis_name="s",
                               num_cores=_NC, num_subcores=_NS)


@functools.partial(
    pl.kernel,
    out_type=jax.ShapeDtypeStruct((_BATCH, _DIM), jnp.float32),
    mesh=_MESH,
    compiler_params=pltpu.CompilerParams(use_tc_tiling_on_sc=False),
    scratch_types=[
        pltpu.VMEM((_N_CHANNELS, _BPW), jnp.float32),
        pltpu.VMEM((_NIDX, _BPW), jnp.int32),
        pltpu.VMEM((_BPW, _DIM), jnp.float32),
        pltpu.SemaphoreType.DMA,
        pltpu.SemaphoreType.DMA,
    ],
)
def _emb(ft_hbm, w_hbm, out_hbm, feat_v, idx_v, acc_v, sem_f, sem0):
    wid = lax.axis_index("s") * _NC + lax.axis_index("c")
    base = wid * _BPW

    pltpu.async_copy(ft_hbm.at[:, pl.ds(base, _BPW)], feat_v, sem_f).wait()

    zeros = jnp.zeros((_L,), jnp.float32)

    @pl.loop(0, _N_CHANNELS)
    def _phase_idx(c):
        for i, r in enumerate(_RESOLUTIONS):
            off = c * (r + 1) + _OFFSETS[i]
            row = i * _N_CHANNELS + c
            for m in range(_MB):
                v = feat_v[c, pl.ds(m * _L, _L)]
                idx_v[row, pl.ds(m * _L, _L)] = _bucket_ids(v, r) + off

    @pl.loop(0, _BPW, unroll=8)
    def _phase_zero(b):
        for q in range(_DIM // _L):
            acc_v[b, pl.ds(q * _L, _L)] = zeros

    def _fire(j):
        pltpu.async_copy(w_hbm.at[idx_v.at[j]], acc_v, sem0, add=True)

    def _wait_one():
        pltpu.make_async_copy(w_hbm.at[idx_v.at[0]], acc_v, sem0).wait()

    for j in range(_KWIN):
        _fire(j)

    @pl.loop(_KWIN, _NIDX)
    def _phase_gather(j):
        _wait_one()
        _fire(j)

    for _ in range(_KWIN):
        _wait_one()

    pltpu.sync_copy(acc_v, out_hbm.at[pl.ds(base, _BPW), :])


def kernel(features, W):
    return _emb(features.T, W)


# speedup vs baseline: 262.0369x; 1.0918x over previous
import functools

import jax
import jax.numpy as jnp
from jax import lax
from jax.experimental import pallas as pl
from jax.experimental.pallas import tpu as pltpu
from jax.experimental.pallas import tpu_sc as plsc

_N_CHANNELS = 100
_RESOLUTIONS = (16, 64, 256, 1024)
_OFFSETS = (0, 1700, 8200, 33900)
_DIM = 64
_BATCH = 4096
_NC, _NS, _L = 2, 16, 16
_NW = _NC * _NS
_BPW = _BATCH // _NW
_NIDX = _N_CHANNELS * len(_RESOLUTIONS)
_MB = _BPW // _L
_KC = 2


def _bucket_ids(v, r):
    _ONE = jnp.full((_L,), 1, jnp.int32)
    _ZERO = jnp.full((_L,), 0, jnp.int32)
    scale = jnp.float32(r - 1)
    delta = jnp.float32(1.0) / jnp.float32(r - 1)
    t = v * scale
    f = t.astype(jnp.int32)
    k = f + jnp.where(t > f.astype(jnp.float32), _ONE, _ZERO)
    k = jnp.minimum(k, r - 1)
    km1 = k - 1
    b_lo = jnp.where(km1 < 0, jnp.float32(-1.0),
                     km1.astype(jnp.float32) * delta)
    b_hi = jnp.where(k == r - 1, jnp.float32(1.0),
                     k.astype(jnp.float32) * delta)
    return (km1 + jnp.where(b_lo < v, _ONE, _ZERO)
            + jnp.where(b_hi < v, _ONE, _ZERO))


_MESH = plsc.VectorSubcoreMesh(core_axis_name="c", subcore_axis_name="s",
                               num_cores=_NC, num_subcores=_NS)


@functools.partial(
    pl.kernel,
    out_type=jax.ShapeDtypeStruct((_BATCH, _DIM), jnp.float32),
    mesh=_MESH,
    compiler_params=pltpu.CompilerParams(use_tc_tiling_on_sc=False),
    scratch_types=[
        pltpu.VMEM((_N_CHANNELS, _BPW), jnp.float32),
        pltpu.VMEM((_NIDX, _BPW), jnp.int32),
        pltpu.VMEM((_BPW, _DIM), jnp.float32),
        pltpu.SemaphoreType.DMA,
        pltpu.SemaphoreType.DMA,
    ],
)
def _emb(ft_hbm, w_hbm, out_hbm, feat_v, idx_v, acc_v, sem_f, sem0):
    wid = lax.axis_index("s") * _NC + lax.axis_index("c")
    base = wid * _BPW

    pltpu.async_copy(ft_hbm.at[:, pl.ds(base, _BPW)], feat_v, sem_f).wait()

    zeros = jnp.zeros((_L,), jnp.float32)

    @pl.loop(0, _BPW, unroll=8)
    def _phase_zero(b):
        for q in range(_DIM // _L):
            acc_v[b, pl.ds(q * _L, _L)] = zeros

    def _fire(j):
        pltpu.async_copy(w_hbm.at[idx_v.at[j]], acc_v, sem0, add=True)

    def _wait_one():
        pltpu.make_async_copy(w_hbm.at[idx_v.at[0]], acc_v, sem0).wait()

    @pl.loop(0, _N_CHANNELS)
    def _phase_main(c):
        for i, r in enumerate(_RESOLUTIONS):
            off = c * (r + 1) + _OFFSETS[i]
            row = i * _N_CHANNELS + c
            for m in range(_MB):
                v = feat_v[c, pl.ds(m * _L, _L)]
                idx_v[row, pl.ds(m * _L, _L)] = _bucket_ids(v, r) + off

        @pl.when(c >= _KC)
        def _():
            for _i in range(len(_RESOLUTIONS)):
                _wait_one()

        for i in range(len(_RESOLUTIONS)):
            _fire(i * _N_CHANNELS + c)

    @pl.loop(0, _KC * len(_RESOLUTIONS))
    def _phase_drain(_j):
        _wait_one()

    pltpu.sync_copy(acc_v, out_hbm.at[pl.ds(base, _BPW), :])


def kernel(features, W):
    return _emb(features.T, W)


# speedup vs baseline: 276.9195x; 1.0568x over previous
import functools

import jax
import jax.numpy as jnp
from jax import lax
from jax.experimental import pallas as pl
from jax.experimental.pallas import tpu as pltpu
from jax.experimental.pallas import tpu_sc as plsc

_N_CHANNELS = 100
_RESOLUTIONS = (16, 64, 256, 1024)
_OFFSETS = (0, 1700, 8200, 33900)
_DIM = 64
_BATCH = 4096
_NC, _NS, _L = 2, 16, 16
_NW = _NC * _NS
_BPW = _BATCH // _NW
_NIDX = _N_CHANNELS * len(_RESOLUTIONS)
_MB = _BPW // _L
_KC = 4


def _bucket_ids(v, r):
    _ONE = jnp.full((_L,), 1, jnp.int32)
    _ZERO = jnp.full((_L,), 0, jnp.int32)
    scale = jnp.float32(r - 1)
    delta = jnp.float32(1.0) / jnp.float32(r - 1)
    t = v * scale
    f = t.astype(jnp.int32)
    k = f + jnp.where(t > f.astype(jnp.float32), _ONE, _ZERO)
    k = jnp.minimum(k, r - 1)
    km1 = k - 1
    b_lo = jnp.where(km1 < 0, jnp.float32(-1.0),
                     km1.astype(jnp.float32) * delta)
    b_hi = jnp.where(k == r - 1, jnp.float32(1.0),
                     k.astype(jnp.float32) * delta)
    return (km1 + jnp.where(b_lo < v, _ONE, _ZERO)
            + jnp.where(b_hi < v, _ONE, _ZERO))


_MESH = plsc.VectorSubcoreMesh(core_axis_name="c", subcore_axis_name="s",
                               num_cores=_NC, num_subcores=_NS)


@functools.partial(
    pl.kernel,
    out_type=jax.ShapeDtypeStruct((_BATCH, _DIM), jnp.float32),
    mesh=_MESH,
    compiler_params=pltpu.CompilerParams(use_tc_tiling_on_sc=False),
    scratch_types=[
        pltpu.VMEM((_N_CHANNELS, _BPW), jnp.float32),
        pltpu.VMEM((_NIDX, _BPW), jnp.int32),
        pltpu.VMEM((_BPW, _DIM), jnp.float32),
        pltpu.SemaphoreType.DMA,
        pltpu.SemaphoreType.DMA,
    ],
)
def _emb(ft_hbm, w_hbm, out_hbm, feat_v, idx_v, acc_v, sem_f, sem0):
    wid = lax.axis_index("s") * _NC + lax.axis_index("c")
    base = wid * _BPW

    pltpu.async_copy(ft_hbm.at[:, pl.ds(base, _BPW)], feat_v, sem_f).wait()

    zeros = jnp.zeros((_L,), jnp.float32)

    @pl.loop(0, _BPW, unroll=8)
    def _phase_zero(b):
        for q in range(_DIM // _L):
            acc_v[b, pl.ds(q * _L, _L)] = zeros

    def _fire(j):
        pltpu.async_copy(w_hbm.at[idx_v.at[j]], acc_v, sem0, add=True)

    def _wait_one():
        pltpu.make_async_copy(w_hbm.at[idx_v.at[0]], acc_v, sem0).wait()

    @pl.loop(0, _N_CHANNELS)
    def _phase_main(c):
        for i, r in enumerate(_RESOLUTIONS):
            off = c * (r + 1) + _OFFSETS[i]
            row = i * _N_CHANNELS + c
            for m in range(_MB):
                v = feat_v[c, pl.ds(m * _L, _L)]
                idx_v[row, pl.ds(m * _L, _L)] = _bucket_ids(v, r) + off

        @pl.when(c >= _KC)
        def _():
            for _i in range(len(_RESOLUTIONS)):
                _wait_one()

        for i in range(len(_RESOLUTIONS)):
            _fire(i * _N_CHANNELS + c)

    @pl.loop(0, _KC * len(_RESOLUTIONS))
    def _phase_drain(_j):
        _wait_one()

    pltpu.sync_copy(acc_v, out_hbm.at[pl.ds(base, _BPW), :])


def kernel(features, W):
    return _emb(features.T, W)
